# baseline (speedup 1.0000x reference)
import functools

import numpy as np
import jax
import jax.numpy as jnp
from jax import lax
from jax.experimental import pallas as pl
from jax.experimental.pallas import tpu as pltpu
from jax.experimental.pallas import tpu_sc as plsc

HIDDEN = 1024
NC = 2
NS = 16
L = 16
NW = NC * NS

CHUNK = 16
NBUF = 6
LOOKAHEAD = 3


def _pos_row(position: int, hidden: int) -> np.ndarray:
    angles = position / np.power(10000.0, np.arange(0, hidden, 2) / hidden)
    row = np.zeros((hidden,), dtype=np.float32)
    row[0::2] = np.sin(angles)
    row[1::2] = np.cos(angles)
    return row


def _make_sc_call(batch: int, seq: int, hidden: int):
    n_rows = batch * seq
    per_w = n_rows // NW
    nchunk = per_w // CHUNK
    w_per_row = seq // per_w
    scale = float(np.sqrt(np.float32(hidden)))
    col_groups = hidden // L

    mesh = plsc.VectorSubcoreMesh(core_axis_name="c", subcore_axis_name="s")

    @functools.partial(
        pl.kernel,
        out_type=jax.ShapeDtypeStruct((n_rows, hidden), jnp.float32),
        mesh=mesh,
        scratch_types=[
            pltpu.VMEM((per_w,), jnp.int32),
            pltpu.VMEM((hidden,), jnp.float32),
        ]
        + [pltpu.VMEM((CHUNK, hidden), jnp.float32) for _ in range(NBUF)]
        + [pltpu.SemaphoreType.DMA for _ in range(2 * NBUF)],
    )
    def run(table_hbm, idx_hbm, pos_hbm, out_hbm, idx_v, pos_v, *rest):
        bufs = rest[:NBUF]
        gsems = rest[NBUF:2 * NBUF]
        ssems = rest[2 * NBUF:]

        wid = lax.axis_index("s") * NC + lax.axis_index("c")
        base = wid * per_w
        row = wid // w_per_row
        col = (wid % w_per_row) * per_w

        pltpu.sync_copy(idx_hbm.at[row, pl.ds(col, per_w)], idx_v)
        pltpu.sync_copy(pos_hbm, pos_v)

        def compute(buf):
            @plsc.parallel_loop(0, col_groups, 1, unroll=2)
            def col_body(j):
                off = j * L
                p = pos_v[pl.ds(off, L)]
                for i in range(CHUNK):
                    buf[i, pl.ds(off, L)] = buf[i, pl.ds(off, L)] * scale + p

        def gather(k):
            return pltpu.async_copy(
                table_hbm.at[idx_v.at[pl.ds(k * CHUNK, CHUNK)]],
                bufs[k % NBUF], gsems[k % NBUF])

        gathers = [None] * nchunk
        scatters = [None] * nchunk
        for k in range(min(LOOKAHEAD, nchunk)):
            gathers[k] = gather(k)

        for k in range(nchunk):
            b = k % NBUF
            if k + LOOKAHEAD < nchunk:
                prev = k + LOOKAHEAD - NBUF
                if prev >= 0:
                    scatters[prev].wait()
                gathers[k + LOOKAHEAD] = gather(k + LOOKAHEAD)
            gathers[k].wait()
            compute(bufs[b])
            scatters[k] = pltpu.async_copy(
                bufs[b], out_hbm.at[pl.ds(base + k * CHUNK, CHUNK)], ssems[b])

        for k in range(max(0, nchunk - NBUF), nchunk):
            scatters[k].wait()

    return run


def kernel(input_ids, emb_table):
    B, T = input_ids.shape
    V, D = emb_table.shape
    ids = input_ids.astype(jnp.int32)
    pos = jnp.asarray(_pos_row(T, D))

    run = _make_sc_call(B, T, D)
    out = run(emb_table, ids, pos)
    return out.reshape(B, T, D)


# speedup vs baseline: 1.1562x; 1.1562x over previous
import functools

import numpy as np
import jax
import jax.numpy as jnp
from jax import lax
from jax.experimental import pallas as pl
from jax.experimental.pallas import tpu as pltpu
from jax.experimental.pallas import tpu_sc as plsc

HIDDEN = 1024
NC = 2
NS = 16
L = 16
NW = NC * NS

CHUNK = 16
NBUF = 6
LOOKAHEAD = 3


def _pos_row(position: int, hidden: int) -> np.ndarray:
    angles = position / np.power(10000.0, np.arange(0, hidden, 2) / hidden)
    row = np.zeros((hidden,), dtype=np.float32)
    row[0::2] = np.sin(angles)
    row[1::2] = np.cos(angles)
    return row


def _make_sc_call(batch: int, seq: int, hidden: int):
    n_rows = batch * seq
    per_w = n_rows // NW
    nchunk = per_w // CHUNK
    w_per_row = seq // per_w
    scale = float(np.sqrt(np.float32(hidden)))
    col_groups = hidden // L

    mesh = plsc.VectorSubcoreMesh(core_axis_name="c", subcore_axis_name="s")

    @functools.partial(
        pl.kernel,
        out_type=jax.ShapeDtypeStruct((n_rows, hidden), jnp.float32),
        mesh=mesh,
        scratch_types=[
            pltpu.VMEM((per_w,), jnp.int32),
            pltpu.VMEM((hidden,), jnp.float32),
        ]
        + [pltpu.VMEM((CHUNK, hidden), jnp.float32) for _ in range(NBUF)]
        + [pltpu.SemaphoreType.DMA for _ in range(2 * NBUF)],
    )
    def run(table_hbm, idx_hbm, pos_hbm, out_hbm, idx_v, pos_v, *rest):
        bufs = rest[:NBUF]
        gsems = rest[NBUF:2 * NBUF]
        ssems = rest[2 * NBUF:]

        wid = lax.axis_index("s") * NC + lax.axis_index("c")
        base = wid * per_w
        row = wid // w_per_row
        col = (wid % w_per_row) * per_w

        pltpu.sync_copy(idx_hbm.at[row, pl.ds(col, per_w)], idx_v)
        pltpu.sync_copy(pos_hbm, pos_v)

        def compute(buf):
            @plsc.parallel_loop(0, col_groups, 1, unroll=1)
            def col_body(j):
                off = j * L
                p = pos_v[pl.ds(off, L)]
                for i in range(CHUNK):
                    buf[i, pl.ds(off, L)] = buf[i, pl.ds(off, L)] * scale + p

        def gather(k):
            return pltpu.async_copy(
                table_hbm.at[idx_v.at[pl.ds(k * CHUNK, CHUNK)]],
                bufs[k % NBUF], gsems[k % NBUF])

        gathers = [None] * nchunk
        scatters = [None] * nchunk
        for k in range(min(LOOKAHEAD, nchunk)):
            gathers[k] = gather(k)

        for k in range(nchunk):
            b = k % NBUF
            if k + LOOKAHEAD < nchunk:
                prev = k + LOOKAHEAD - NBUF
                if prev >= 0:
                    scatters[prev].wait()
                gathers[k + LOOKAHEAD] = gather(k + LOOKAHEAD)
            gathers[k].wait()
            compute(bufs[b])
            scatters[k] = pltpu.async_copy(
                bufs[b], out_hbm.at[pl.ds(base + k * CHUNK, CHUNK)], ssems[b])

        for k in range(max(0, nchunk - NBUF), nchunk):
            scatters[k].wait()

    return run


def kernel(input_ids, emb_table):
    B, T = input_ids.shape
    V, D = emb_table.shape
    ids = input_ids.astype(jnp.int32)
    pos = jnp.asarray(_pos_row(T, D))

    run = _make_sc_call(B, T, D)
    out = run(emb_table, ids, pos)
    return out.reshape(B, T, D)


# speedup vs baseline: 1.2044x; 1.0417x over previous
import functools

import numpy as np
import jax
import jax.numpy as jnp
from jax import lax
from jax.experimental import pallas as pl
from jax.experimental.pallas import tpu as pltpu
from jax.experimental.pallas import tpu_sc as plsc

HIDDEN = 1024
NC = 2
NS = 16
L = 16
NW = NC * NS

CHUNK = 32
NBUF = 3
LOOKAHEAD = 2


def _pos_row(position: int, hidden: int) -> np.ndarray:
    angles = position / np.power(10000.0, np.arange(0, hidden, 2) / hidden)
    row = np.zeros((hidden,), dtype=np.float32)
    row[0::2] = np.sin(angles)
    row[1::2] = np.cos(angles)
    return row


def _make_sc_call(batch: int, seq: int, hidden: int):
    n_rows = batch * seq
    per_w = n_rows // NW
    nchunk = per_w // CHUNK
    w_per_row = seq // per_w
    scale = float(np.sqrt(np.float32(hidden)))
    col_groups = hidden // L

    mesh = plsc.VectorSubcoreMesh(core_axis_name="c", subcore_axis_name="s")

    @functools.partial(
        pl.kernel,
        out_type=jax.ShapeDtypeStruct((n_rows, hidden), jnp.float32),
        mesh=mesh,
        scratch_types=[
            pltpu.VMEM((per_w,), jnp.int32),
            pltpu.VMEM((hidden,), jnp.float32),
        ]
        + [pltpu.VMEM((CHUNK, hidden), jnp.float32) for _ in range(NBUF)]
        + [pltpu.SemaphoreType.DMA for _ in range(2 * NBUF)],
    )
    def run(table_hbm, idx_hbm, pos_hbm, out_hbm, idx_v, pos_v, *rest):
        bufs = rest[:NBUF]
        gsems = rest[NBUF:2 * NBUF]
        ssems = rest[2 * NBUF:]

        wid = lax.axis_index("s") * NC + lax.axis_index("c")
        base = wid * per_w
        row = wid // w_per_row
        col = (wid % w_per_row) * per_w

        pltpu.sync_copy(idx_hbm.at[row, pl.ds(col, per_w)], idx_v)
        pltpu.sync_copy(pos_hbm, pos_v)

        def compute(buf):
            @plsc.parallel_loop(0, col_groups, 1, unroll=1)
            def col_body(j):
                off = j * L
                p = pos_v[pl.ds(off, L)]
                for i in range(CHUNK):
                    buf[i, pl.ds(off, L)] = buf[i, pl.ds(off, L)] * scale + p

        def gather(k):
            return pltpu.async_copy(
                table_hbm.at[idx_v.at[pl.ds(k * CHUNK, CHUNK)]],
                bufs[k % NBUF], gsems[k % NBUF])

        gathers = [None] * nchunk
        scatters = [None] * nchunk
        for k in range(min(LOOKAHEAD, nchunk)):
            gathers[k] = gather(k)

        for k in range(nchunk):
            b = k % NBUF
            if k + LOOKAHEAD < nchunk:
                prev = k + LOOKAHEAD - NBUF
                if prev >= 0:
                    scatters[prev].wait()
                gathers[k + LOOKAHEAD] = gather(k + LOOKAHEAD)
            gathers[k].wait()
            compute(bufs[b])
            scatters[k] = pltpu.async_copy(
                bufs[b], out_hbm.at[pl.ds(base + k * CHUNK, CHUNK)], ssems[b])

        for k in range(max(0, nchunk - NBUF), nchunk):
            scatters[k].wait()

    return run


def kernel(input_ids, emb_table):
    B, T = input_ids.shape
    V, D = emb_table.shape
    ids = input_ids.astype(jnp.int32)
    pos = jnp.asarray(_pos_row(T, D))

    run = _make_sc_call(B, T, D)
    out = run(emb_table, ids, pos)
    return out.reshape(B, T, D)


# speedup vs baseline: 1.2331x; 1.0239x over previous
import functools

import numpy as np
import jax
import jax.numpy as jnp
from jax import lax
from jax.experimental import pallas as pl
from jax.experimental.pallas import tpu as pltpu
from jax.experimental.pallas import tpu_sc as plsc

HIDDEN = 1024
NC = 2
NS = 16
L = 16
NW = NC * NS

CHUNK = 32
NBUF = 3
LOOKAHEAD = 2


def _pos_row(position: int, hidden: int) -> np.ndarray:
    angles = position / np.power(10000.0, np.arange(0, hidden, 2) / hidden)
    row = np.zeros((hidden,), dtype=np.float32)
    row[0::2] = np.sin(angles)
    row[1::2] = np.cos(angles)
    return row


def _make_sc_call(batch: int, seq: int, hidden: int):
    n_rows = batch * seq
    per_w = n_rows // NW
    nchunk = per_w // CHUNK
    w_per_row = seq // per_w
    scale = float(np.sqrt(np.float32(hidden)))
    col_groups = hidden // L

    mesh = plsc.VectorSubcoreMesh(core_axis_name="c", subcore_axis_name="s")

    @functools.partial(
        pl.kernel,
        out_type=jax.ShapeDtypeStruct((n_rows, hidden), jnp.float32),
        mesh=mesh,
        scratch_types=[
            pltpu.VMEM((per_w,), jnp.int32),
            pltpu.VMEM((hidden,), jnp.float32),
        ]
        + [pltpu.VMEM((CHUNK, hidden), jnp.float32) for _ in range(NBUF)]
        + [pltpu.SemaphoreType.DMA for _ in range(2 * NBUF + 1)],
    )
    def run(table_hbm, idx_hbm, pos_hbm, out_hbm, idx_v, pos_v, *rest):
        bufs = rest[:NBUF]
        gsems = rest[NBUF:2 * NBUF]
        ssems = rest[2 * NBUF:3 * NBUF]
        psem = rest[3 * NBUF]

        wid = lax.axis_index("s") * NC + lax.axis_index("c")
        base = wid * per_w
        row = wid // w_per_row
        col = (wid % w_per_row) * per_w

        pos_cp = pltpu.async_copy(pos_hbm, pos_v, psem)
        pltpu.sync_copy(idx_hbm.at[row, pl.ds(col, per_w)], idx_v)

        def compute(buf):
            @plsc.parallel_loop(0, col_groups, 1, unroll=1)
            def col_body(j):
                off = j * L
                p = pos_v[pl.ds(off, L)]
                for i in range(CHUNK):
                    buf[i, pl.ds(off, L)] = buf[i, pl.ds(off, L)] * scale + p

        def gather(k):
            return pltpu.async_copy(
                table_hbm.at[idx_v.at[pl.ds(k * CHUNK, CHUNK)]],
                bufs[k % NBUF], gsems[k % NBUF])

        gathers = [None] * nchunk
        scatters = [None] * nchunk
        for k in range(min(LOOKAHEAD, nchunk)):
            gathers[k] = gather(k)
        pos_cp.wait()

        for k in range(nchunk):
            b = k % NBUF
            if k + LOOKAHEAD < nchunk:
                prev = k + LOOKAHEAD - NBUF
                if prev >= 0:
                    scatters[prev].wait()
                gathers[k + LOOKAHEAD] = gather(k + LOOKAHEAD)
            gathers[k].wait()
            compute(bufs[b])
            scatters[k] = pltpu.async_copy(
                bufs[b], out_hbm.at[pl.ds(base + k * CHUNK, CHUNK)], ssems[b])

        for k in range(max(0, nchunk - NBUF), nchunk):
            scatters[k].wait()

    return run


def kernel(input_ids, emb_table):
    B, T = input_ids.shape
    V, D = emb_table.shape
    ids = input_ids.astype(jnp.int32)
    pos = jnp.asarray(_pos_row(T, D))

    run = _make_sc_call(B, T, D)
    out = run(emb_table, ids, pos)
    return out.reshape(B, T, D)
